# baseline (speedup 1.0000x reference)
import functools

import jax
import jax.numpy as jnp
from jax import lax
from jax.experimental import pallas as pl
from jax.experimental.pallas import tpu as pltpu
from jax.experimental.pallas import tpu_sc as plsc

BATCH = 16384
D = 64
VOCAB = 1000000

_info = plsc.get_sparse_core_info()
NC, NS, L = _info.num_cores, _info.num_subcores, _info.num_lanes
NW = NC * NS
BPW = BATCH // NW

NBLK = (VOCAB + 127) // 128
LASTK = NBLK - 1
NJ = 245
CAP = 16
LCAP = 768
NSL = BATCH // L
DUMP = BATCH
STAGE_ROWS = BATCH + 8
SW = 128


def _filter_ids(allids_v, loc_ids, loc_pos, widv, lanes):
    def body(s, goffv):
        ids = allids_v[pl.ds(s * L, L)]
        mine = ((ids >> 7) & (NW - 1)) == widv
        csum = plsc.cumsum(mine.astype(jnp.int32))
        slot = goffv + csum - 1
        keep = mine & (slot >= 0) & (slot < LCAP)
        slot = jnp.where(keep, slot, 0)
        plsc.store_scatter(loc_ids, [slot], ids, mask=keep)
        plsc.store_scatter(loc_pos, [slot], s * L + lanes, mask=keep)
        return goffv + plsc.all_reduce_population_count(mine)
    return lax.fori_loop(0, NSL, body, jnp.zeros((L,), jnp.int32))


def _bin_ids(loc_ids, loc_pos, b_ids, b_pos, counts_v, goffv, lanes):
    def body(s2, carry):
        ids = loc_ids[pl.ds(s2 * L, L)]
        pos = loc_pos[pl.ds(s2 * L, L)]
        valid = (s2 * L + lanes) < goffv
        vi = valid.astype(jnp.int32)
        blkj = jnp.where(valid, ids >> 12, 0)
        rank = jnp.zeros((L,), jnp.int32)
        haslater = jnp.zeros((L,), jnp.int32)
        for s in range(1, L):
            idxb = jnp.maximum(lanes - s, 0)
            eqb = (jnp.take(blkj, idxb) == blkj) & (lanes >= s)
            rank = rank + (eqb & (jnp.take(vi, idxb) > 0)).astype(jnp.int32)
            idxf = jnp.minimum(lanes + s, L - 1)
            eqf = (jnp.take(blkj, idxf) == blkj) & (lanes + s <= L - 1)
            haslater = haslater + (eqf & (jnp.take(vi, idxf) > 0)).astype(jnp.int32)
        base = plsc.load_gather(counts_v, [blkj])
        slot = base + rank
        keep = valid & (slot < CAP)
        slot_s = jnp.where(keep, slot, 0)
        plsc.store_scatter(b_ids, [blkj, slot_s], ids, mask=keep)
        plsc.store_scatter(b_pos, [blkj, slot_s], pos, mask=keep)
        islast = valid & (haslater == 0)
        plsc.store_scatter(counts_v, [blkj], jnp.minimum(slot + 1, CAP),
                           mask=islast)
        return carry
    lax.fori_loop(0, LCAP // L, body, 0)


def _one_table_pass(ids_hbm, tab_hbm, stage_hbm, wid, widv, lanes,
                    allids_v, loc_ids, loc_pos, b_ids, b_pos, counts_v,
                    winbuf, rowtmp, posmat, wsems, ssems):
    pltpu.sync_copy(ids_hbm, allids_v)
    for i in range(256 // L):
        counts_v[pl.ds(i * L, L)] = jnp.zeros((L,), jnp.int32)
    goffv = _filter_ids(allids_v, loc_ids, loc_pos, widv, lanes)
    _bin_ids(loc_ids, loc_pos, b_ids, b_pos, counts_v, goffv, lanes)

    def fire(jf, b):
        kf = jnp.minimum(wid + 32 * jf, LASTK)
        koff = pl.multiple_of(kf * 128, 128)
        pltpu.async_copy(tab_hbm.at[:, pl.ds(koff, 128)],
                         winbuf.at[b], wsems[b])

    def drain_win(b):
        pltpu.make_async_copy(tab_hbm.at[:, pl.ds(0, 128)],
                              winbuf.at[b], wsems[b]).wait()

    def drain_scat(b):
        pltpu.make_async_copy(stage_hbm.at[pl.ds(0, CAP)],
                              rowtmp.at[b], ssems[b]).wait()

    def extract(j, b):
        jb = jnp.full((L,), 0, jnp.int32) + j
        cnt = plsc.load_gather(counts_v, [jb])
        ids16 = b_ids[j]
        pos16 = b_pos[j]
        rm = ids16 & 127
        posmat[b] = jnp.where(lanes < cnt, pos16, DUMP)
        for j2 in range(CAP):
            rmb = jnp.take(rm, jnp.full((L,), j2, jnp.int32))
            for q in range(D // L):
                vals = plsc.load_gather(winbuf.at[b], [q * L + lanes, rmb])
                rowtmp[b, j2, pl.ds(q * L, L)] = vals
        pltpu.async_copy(rowtmp.at[b], stage_hbm.at[posmat.at[b]], ssems[b])

    fire(0, 0)
    fire(1, 1)

    def outer(j2, carry):
        for b in range(2):
            j = 2 * j2 + b

            @pl.when(j < NJ)
            def _():
                drain_win(b)

                @pl.when(j >= 2)
                def _():
                    drain_scat(b)

                extract(j, b)

                @pl.when(j + 2 < NJ)
                def _():
                    fire(j + 2, b)
        return carry

    lax.fori_loop(0, (NJ + 1) // 2, outer, 0)
    drain_scat(0)
    drain_scat(1)


@functools.partial(
    pl.kernel,
    out_type=(jax.ShapeDtypeStruct((STAGE_ROWS, SW), jnp.float32),
              jax.ShapeDtypeStruct((STAGE_ROWS, SW), jnp.float32)),
    mesh=plsc.VectorSubcoreMesh(core_axis_name="c", subcore_axis_name="s"),
    compiler_params=pltpu.CompilerParams(needs_layout_passes=False),
    scratch_types=[
        pltpu.VMEM((BATCH,), jnp.int32),
        pltpu.VMEM((LCAP,), jnp.int32),
        pltpu.VMEM((LCAP,), jnp.int32),
        pltpu.VMEM((NJ, CAP), jnp.int32),
        pltpu.VMEM((NJ, CAP), jnp.int32),
        pltpu.VMEM((256,), jnp.int32),
        pltpu.VMEM((2, D, 128), jnp.float32),
        pltpu.VMEM((2, CAP, SW), jnp.float32),
        pltpu.VMEM((2, CAP), jnp.int32),
        pltpu.SemaphoreType.DMA,
        pltpu.SemaphoreType.DMA,
        pltpu.SemaphoreType.DMA,
        pltpu.SemaphoreType.DMA,
    ],
)
def _extract_kernel(uid_hbm, iid_hbm, ut_hbm, it_hbm, ustage, istage,
                    allids_v, loc_ids, loc_pos, b_ids, b_pos, counts_v,
                    winbuf, rowtmp, posmat, ws0, ws1, ss0, ss1):
    wid = lax.axis_index("s") * NC + lax.axis_index("c")
    widv = jnp.full((L,), 0, jnp.int32) + wid
    lanes = lax.broadcasted_iota(jnp.int32, (L,), 0)
    _one_table_pass(uid_hbm, ut_hbm, ustage, wid, widv, lanes,
                    allids_v, loc_ids, loc_pos, b_ids, b_pos, counts_v,
                    winbuf, rowtmp, posmat, (ws0, ws1), (ss0, ss1))
    _one_table_pass(iid_hbm, it_hbm, istage, wid, widv, lanes,
                    allids_v, loc_ids, loc_pos, b_ids, b_pos, counts_v,
                    winbuf, rowtmp, posmat, (ws0, ws1), (ss0, ss1))


@functools.partial(
    pl.kernel,
    out_type=jax.ShapeDtypeStruct((BATCH,), jnp.float32),
    mesh=plsc.VectorSubcoreMesh(core_axis_name="c", subcore_axis_name="s"),
    compiler_params=pltpu.CompilerParams(needs_layout_passes=False),
    scratch_types=[
        pltpu.VMEM((128, SW), jnp.float32),
        pltpu.VMEM((128, SW), jnp.float32),
        pltpu.VMEM((BPW,), jnp.float32),
    ],
)
def _dot_kernel(ustage, istage, out_hbm, uv, iv, out_v):
    wid = lax.axis_index("s") * NC + lax.axis_index("c")
    base = wid * BPW
    lanes = lax.broadcasted_iota(jnp.int32, (L,), 0)

    for c in range(BPW // 128):
        pltpu.sync_copy(ustage.at[pl.ds(base + c * 128, 128)], uv)
        pltpu.sync_copy(istage.at[pl.ds(base + c * 128, 128)], iv)
        for g in range(128 // L):
            rows = g * L + lanes
            acc = jnp.zeros((L,), jnp.float32)
            for d in range(D):
                dv = jnp.full((L,), d, jnp.int32)
                acc = acc + (plsc.load_gather(uv, [rows, dv])
                             * plsc.load_gather(iv, [rows, dv]))
            out_v[pl.ds(c * 128 + g * L, L)] = acc

    pltpu.sync_copy(out_v, out_hbm.at[pl.ds(base, BPW)])


def kernel(user_ids, item_ids, user_table, item_table):
    ustage, istage = _extract_kernel(
        user_ids.astype(jnp.int32), item_ids.astype(jnp.int32),
        user_table.T, item_table.T)
    return _dot_kernel(ustage, istage)


# speedup vs baseline: 15.6688x; 15.6688x over previous
import functools

import jax
import jax.numpy as jnp
from jax import lax
from jax.experimental import pallas as pl
from jax.experimental.pallas import tpu as pltpu
from jax.experimental.pallas import tpu_sc as plsc

BATCH = 16384
D = 64
VOCAB = 1000000

_info = plsc.get_sparse_core_info()
NC, NS, L = _info.num_cores, _info.num_subcores, _info.num_lanes
NW = NC * NS
BPW = BATCH // NW

NBLK = (VOCAB + 127) // 128
LASTK = NBLK - 1
NJ = 245
CAP = 16
LCAP = 768
NSL = BATCH // L
DUMPBASE = BATCH
STAGE_ROWS = BATCH + NW * L
SW = 128


def _filter_ids(allids_v, loc_ids, loc_pos, widv, lanes):
    def body(s, goffv):
        ids = allids_v[pl.ds(s * L, L)]
        mine = ((ids >> 7) & (NW - 1)) == widv
        csum = plsc.cumsum(mine.astype(jnp.int32))
        slot = goffv + csum - 1
        keep = mine & (slot >= 0) & (slot < LCAP)
        slot = jnp.where(keep, slot, 0)
        plsc.store_scatter(loc_ids, [slot], ids, mask=keep)
        plsc.store_scatter(loc_pos, [slot], s * L + lanes, mask=keep)
        return goffv + plsc.all_reduce_population_count(mine)
    return lax.fori_loop(0, NSL, body, jnp.zeros((L,), jnp.int32))


def _bin_ids(loc_ids, loc_pos, b_ids, b_pos, counts_v, goffv, lanes):
    def body(s2, carry):
        ids = loc_ids[pl.ds(s2 * L, L)]
        pos = loc_pos[pl.ds(s2 * L, L)]
        valid = (s2 * L + lanes) < goffv
        vi = valid.astype(jnp.int32)
        blkj = jnp.where(valid, ids >> 12, 0)
        rank = jnp.zeros((L,), jnp.int32)
        haslater = jnp.zeros((L,), jnp.int32)
        for s in range(1, L):
            idxb = jnp.maximum(lanes - s, 0)
            eqb = (jnp.take(blkj, idxb) == blkj) & (lanes >= s)
            rank = rank + (eqb & (jnp.take(vi, idxb) > 0)).astype(jnp.int32)
            idxf = jnp.minimum(lanes + s, L - 1)
            eqf = (jnp.take(blkj, idxf) == blkj) & (lanes + s <= L - 1)
            haslater = haslater + (eqf & (jnp.take(vi, idxf) > 0)).astype(jnp.int32)
        base = plsc.load_gather(counts_v, [blkj])
        slot = base + rank
        keep = valid & (slot < CAP)
        slot_s = jnp.where(keep, slot, 0)
        plsc.store_scatter(b_ids, [blkj, slot_s], ids, mask=keep)
        plsc.store_scatter(b_pos, [blkj, slot_s], pos, mask=keep)
        islast = valid & (haslater == 0)
        plsc.store_scatter(counts_v, [blkj], jnp.minimum(slot + 1, CAP),
                           mask=islast)
        return carry
    lax.fori_loop(0, LCAP // L, body, 0)


def _one_table_pass(ids_hbm, tab_hbm, stage_hbm, wid, widv, lanes,
                    allids_v, loc_ids, loc_pos, b_ids, b_pos, counts_v,
                    winbuf, rowtmp, posmat, wsems, ssems):
    pltpu.sync_copy(ids_hbm, allids_v)
    for i in range(256 // L):
        counts_v[pl.ds(i * L, L)] = jnp.zeros((L,), jnp.int32)
    goffv = _filter_ids(allids_v, loc_ids, loc_pos, widv, lanes)
    _bin_ids(loc_ids, loc_pos, b_ids, b_pos, counts_v, goffv, lanes)

    def fire(jf, b):
        kf = jnp.minimum(wid + 32 * jf, LASTK)
        koff = pl.multiple_of(kf * 128, 128)
        pltpu.async_copy(tab_hbm.at[:, pl.ds(koff, 128)],
                         winbuf.at[b], wsems[b])

    def drain_win(b):
        pltpu.make_async_copy(tab_hbm.at[:, pl.ds(0, 128)],
                              winbuf.at[b], wsems[b]).wait()

    def drain_scat(b):
        pltpu.make_async_copy(stage_hbm.at[pl.ds(0, CAP)],
                              rowtmp.at[b], ssems[b]).wait()

    def extract(j, b):
        jb = jnp.full((L,), 0, jnp.int32) + j
        cnt = plsc.load_gather(counts_v, [jb])
        ids16 = b_ids[j]
        pos16 = b_pos[j]
        rm = ids16 & 127
        posmat[b] = jnp.where(lanes < cnt, pos16, DUMPBASE + widv * L + lanes)
        for j2 in range(CAP):
            rmb = jnp.take(rm, jnp.full((L,), j2, jnp.int32))
            for q in range(D // L):
                vals = plsc.load_gather(winbuf.at[b], [q * L + lanes, rmb])
                rowtmp[b, j2, pl.ds(q * L, L)] = vals
        pltpu.async_copy(rowtmp.at[b], stage_hbm.at[posmat.at[b]], ssems[b])

    fire(0, 0)
    fire(1, 1)

    def outer(j2, carry):
        for b in range(2):
            j = 2 * j2 + b

            @pl.when(j < NJ)
            def _():
                drain_win(b)

                @pl.when(j >= 2)
                def _():
                    drain_scat(b)

                extract(j, b)

                @pl.when(j + 2 < NJ)
                def _():
                    fire(j + 2, b)
        return carry

    lax.fori_loop(0, (NJ + 1) // 2, outer, 0)
    drain_scat(0)
    drain_scat(1)


@functools.partial(
    pl.kernel,
    out_type=(jax.ShapeDtypeStruct((STAGE_ROWS, SW), jnp.float32),
              jax.ShapeDtypeStruct((STAGE_ROWS, SW), jnp.float32)),
    mesh=plsc.VectorSubcoreMesh(core_axis_name="c", subcore_axis_name="s"),
    compiler_params=pltpu.CompilerParams(needs_layout_passes=False),
    scratch_types=[
        pltpu.VMEM((BATCH,), jnp.int32),
        pltpu.VMEM((LCAP,), jnp.int32),
        pltpu.VMEM((LCAP,), jnp.int32),
        pltpu.VMEM((NJ, CAP), jnp.int32),
        pltpu.VMEM((NJ, CAP), jnp.int32),
        pltpu.VMEM((256,), jnp.int32),
        pltpu.VMEM((2, D, 128), jnp.float32),
        pltpu.VMEM((2, CAP, SW), jnp.float32),
        pltpu.VMEM((2, CAP), jnp.int32),
        pltpu.SemaphoreType.DMA,
        pltpu.SemaphoreType.DMA,
        pltpu.SemaphoreType.DMA,
        pltpu.SemaphoreType.DMA,
    ],
)
def _extract_kernel(uid_hbm, iid_hbm, ut_hbm, it_hbm, ustage, istage,
                    allids_v, loc_ids, loc_pos, b_ids, b_pos, counts_v,
                    winbuf, rowtmp, posmat, ws0, ws1, ss0, ss1):
    wid = lax.axis_index("s") * NC + lax.axis_index("c")
    widv = jnp.full((L,), 0, jnp.int32) + wid
    lanes = lax.broadcasted_iota(jnp.int32, (L,), 0)
    _one_table_pass(uid_hbm, ut_hbm, ustage, wid, widv, lanes,
                    allids_v, loc_ids, loc_pos, b_ids, b_pos, counts_v,
                    winbuf, rowtmp, posmat, (ws0, ws1), (ss0, ss1))
    _one_table_pass(iid_hbm, it_hbm, istage, wid, widv, lanes,
                    allids_v, loc_ids, loc_pos, b_ids, b_pos, counts_v,
                    winbuf, rowtmp, posmat, (ws0, ws1), (ss0, ss1))


@functools.partial(
    pl.kernel,
    out_type=jax.ShapeDtypeStruct((BATCH,), jnp.float32),
    mesh=plsc.VectorSubcoreMesh(core_axis_name="c", subcore_axis_name="s"),
    compiler_params=pltpu.CompilerParams(needs_layout_passes=False),
    scratch_types=[
        pltpu.VMEM((128, SW), jnp.float32),
        pltpu.VMEM((128, SW), jnp.float32),
        pltpu.VMEM((BPW,), jnp.float32),
    ],
)
def _dot_kernel(ustage, istage, out_hbm, uv, iv, out_v):
    wid = lax.axis_index("s") * NC + lax.axis_index("c")
    base = wid * BPW
    lanes = lax.broadcasted_iota(jnp.int32, (L,), 0)

    for c in range(BPW // 128):
        pltpu.sync_copy(ustage.at[pl.ds(base + c * 128, 128)], uv)
        pltpu.sync_copy(istage.at[pl.ds(base + c * 128, 128)], iv)
        for g in range(128 // L):
            rows = g * L + lanes
            acc = jnp.zeros((L,), jnp.float32)
            for d in range(D):
                dv = jnp.full((L,), d, jnp.int32)
                acc = acc + (plsc.load_gather(uv, [rows, dv])
                             * plsc.load_gather(iv, [rows, dv]))
            out_v[pl.ds(c * 128 + g * L, L)] = acc

    pltpu.sync_copy(out_v, out_hbm.at[pl.ds(base, BPW)])


def kernel(user_ids, item_ids, user_table, item_table):
    ustage, istage = _extract_kernel(
        user_ids.astype(jnp.int32), item_ids.astype(jnp.int32),
        user_table.T, item_table.T)
    return _dot_kernel(ustage, istage)


# speedup vs baseline: 19.5485x; 1.2476x over previous
import functools

import jax
import jax.numpy as jnp
from jax import lax
from jax.experimental import pallas as pl
from jax.experimental.pallas import tpu as pltpu
from jax.experimental.pallas import tpu_sc as plsc

BATCH = 16384
D = 64
VOCAB = 1000000

_info = plsc.get_sparse_core_info()
NC, NS, L = _info.num_cores, _info.num_subcores, _info.num_lanes
NW = NC * NS
BPW = BATCH // NW

NBLK = (VOCAB + 127) // 128
LASTK = NBLK - 1
NJ = 245
CAP = 16
GR = 4
LCAP = 768
NSL = BATCH // L
DUMPBASE = BATCH
STAGE_ROWS = BATCH + NW * L
SW = 128


def _filter_ids(allids_v, loc_ids, loc_pos, widv, lanes):
    def body(s, goffv):
        ids = allids_v[pl.ds(s * L, L)]
        mine = ((ids >> 7) & (NW - 1)) == widv
        csum = plsc.cumsum(mine.astype(jnp.int32))
        slot = goffv + csum - 1
        keep = mine & (slot >= 0) & (slot < LCAP)
        slot = jnp.where(keep, slot, 0)
        plsc.store_scatter(loc_ids, [slot], ids, mask=keep)
        plsc.store_scatter(loc_pos, [slot], s * L + lanes, mask=keep)
        return goffv + plsc.all_reduce_population_count(mine)
    return lax.fori_loop(0, NSL, body, jnp.zeros((L,), jnp.int32))


def _bin_ids(loc_ids, loc_pos, b_ids, b_pos, counts_v, goffv, lanes):
    def body(s2, carry):
        ids = loc_ids[pl.ds(s2 * L, L)]
        pos = loc_pos[pl.ds(s2 * L, L)]
        valid = (s2 * L + lanes) < goffv
        vi = valid.astype(jnp.int32)
        blkj = jnp.where(valid, ids >> 12, 0)
        rank = jnp.zeros((L,), jnp.int32)
        haslater = jnp.zeros((L,), jnp.int32)
        for s in range(1, L):
            idxb = jnp.maximum(lanes - s, 0)
            eqb = (jnp.take(blkj, idxb) == blkj) & (lanes >= s)
            rank = rank + (eqb & (jnp.take(vi, idxb) > 0)).astype(jnp.int32)
            idxf = jnp.minimum(lanes + s, L - 1)
            eqf = (jnp.take(blkj, idxf) == blkj) & (lanes + s <= L - 1)
            haslater = haslater + (eqf & (jnp.take(vi, idxf) > 0)).astype(jnp.int32)
        base = plsc.load_gather(counts_v, [blkj])
        slot = base + rank
        keep = valid & (slot < CAP)
        slot_s = jnp.where(keep, slot, 0)
        plsc.store_scatter(b_ids, [blkj, slot_s], ids, mask=keep)
        plsc.store_scatter(b_pos, [blkj, slot_s], pos, mask=keep)
        islast = valid & (haslater == 0)
        plsc.store_scatter(counts_v, [blkj], jnp.minimum(slot + 1, CAP),
                           mask=islast)
        return carry
    lax.fori_loop(0, LCAP // L, body, 0)


def _one_table_pass(ids_hbm, tab_hbm, stage_hbm, wid, widv, lanes,
                    allids_v, loc_ids, loc_pos, b_ids, b_pos, counts_v,
                    winbuf, rowtmp, posmat, counts_s, wsems, ssems):
    pltpu.sync_copy(ids_hbm, allids_v)
    for i in range(256 // L):
        counts_v[pl.ds(i * L, L)] = jnp.zeros((L,), jnp.int32)
    goffv = _filter_ids(allids_v, loc_ids, loc_pos, widv, lanes)
    _bin_ids(loc_ids, loc_pos, b_ids, b_pos, counts_v, goffv, lanes)

    def fire(jf, b):
        @pl.when(counts_s[jf] > 0)
        def _():
            kf = jnp.minimum(wid + 32 * jf, LASTK)
            koff = pl.multiple_of(kf * 128, 128)
            pltpu.async_copy(tab_hbm.at[:, pl.ds(koff, 128)],
                             winbuf.at[b], wsems[b])

    def drain_win(b):
        pltpu.make_async_copy(tab_hbm.at[:, pl.ds(0, 128)],
                              winbuf.at[b], wsems[b]).wait()

    def drain_scat(jd, b):
        @pl.when(counts_s[jd] > 0)
        def _():
            pltpu.make_async_copy(stage_hbm.at[pl.ds(0, CAP)],
                                  rowtmp.at[b], ssems[b]).wait()

    def extract(j, b, cnt):
        cntv = jnp.full((L,), 0, jnp.int32) + cnt
        ids16 = b_ids[j]
        pos16 = b_pos[j]
        rm = ids16 & 127
        posmat[b] = jnp.where(lanes < cntv, pos16,
                              DUMPBASE + widv * L + lanes)
        for j2 in range(CAP):

            @pl.when(cnt > j2)
            def _():
                rmb = jnp.take(rm, jnp.full((L,), j2, jnp.int32))
                for q in range(D // L):
                    vals = plsc.load_gather(winbuf.at[b], [q * L + lanes, rmb])
                    rowtmp[b, j2, pl.ds(q * L, L)] = vals
        pltpu.async_copy(rowtmp.at[b], stage_hbm.at[posmat.at[b]], ssems[b])

    for i in range(256 // L):
        cv = counts_v[pl.ds(i * L, L)]
        for l in range(L):
            counts_s[i * L + l] = cv[l]

    fire(0, 0)
    fire(1, 1)

    def outer(j2, carry):
        for b in range(2):
            j = 2 * j2 + b

            @pl.when(j < NJ)
            def _():
                cnt = counts_s[j]

                @pl.when(cnt > 0)
                def _():
                    drain_win(b)

                @pl.when(j >= 2)
                def _():
                    drain_scat(j - 2, b)

                @pl.when(cnt > 0)
                def _():
                    extract(j, b, cnt)

                @pl.when(j + 2 < NJ)
                def _():
                    fire(j + 2, b)
        return carry

    lax.fori_loop(0, (NJ + 1) // 2, outer, 0)
    drain_scat(NJ - 2, (NJ - 2) % 2)
    drain_scat(NJ - 1, (NJ - 1) % 2)


@functools.partial(
    pl.kernel,
    out_type=(jax.ShapeDtypeStruct((STAGE_ROWS, SW), jnp.float32),
              jax.ShapeDtypeStruct((STAGE_ROWS, SW), jnp.float32)),
    mesh=plsc.VectorSubcoreMesh(core_axis_name="c", subcore_axis_name="s"),
    compiler_params=pltpu.CompilerParams(needs_layout_passes=False),
    scratch_types=[
        pltpu.VMEM((BATCH,), jnp.int32),
        pltpu.VMEM((LCAP,), jnp.int32),
        pltpu.VMEM((LCAP,), jnp.int32),
        pltpu.VMEM((NJ, CAP), jnp.int32),
        pltpu.VMEM((NJ, CAP), jnp.int32),
        pltpu.VMEM((256,), jnp.int32),
        pltpu.VMEM((2, D, 128), jnp.float32),
        pltpu.VMEM((2, CAP, SW), jnp.float32),
        pltpu.VMEM((2, CAP), jnp.int32),
        pltpu.SMEM((256,), jnp.int32),
        pltpu.SemaphoreType.DMA,
        pltpu.SemaphoreType.DMA,
        pltpu.SemaphoreType.DMA,
        pltpu.SemaphoreType.DMA,
    ],
)
def _extract_kernel(uid_hbm, iid_hbm, ut_hbm, it_hbm, ustage, istage,
                    allids_v, loc_ids, loc_pos, b_ids, b_pos, counts_v,
                    winbuf, rowtmp, posmat, counts_s, ws0, ws1, ss0, ss1):
    wid = lax.axis_index("s") * NC + lax.axis_index("c")
    widv = jnp.full((L,), 0, jnp.int32) + wid
    lanes = lax.broadcasted_iota(jnp.int32, (L,), 0)
    _one_table_pass(uid_hbm, ut_hbm, ustage, wid, widv, lanes,
                    allids_v, loc_ids, loc_pos, b_ids, b_pos, counts_v,
                    winbuf, rowtmp, posmat, counts_s, (ws0, ws1), (ss0, ss1))
    _one_table_pass(iid_hbm, it_hbm, istage, wid, widv, lanes,
                    allids_v, loc_ids, loc_pos, b_ids, b_pos, counts_v,
                    winbuf, rowtmp, posmat, counts_s, (ws0, ws1), (ss0, ss1))


@functools.partial(
    pl.kernel,
    out_type=jax.ShapeDtypeStruct((BATCH,), jnp.float32),
    mesh=plsc.VectorSubcoreMesh(core_axis_name="c", subcore_axis_name="s"),
    compiler_params=pltpu.CompilerParams(needs_layout_passes=False),
    scratch_types=[
        pltpu.VMEM((128, SW), jnp.float32),
        pltpu.VMEM((128, SW), jnp.float32),
        pltpu.VMEM((BPW,), jnp.float32),
    ],
)
def _dot_kernel(ustage, istage, out_hbm, uv, iv, out_v):
    wid = lax.axis_index("s") * NC + lax.axis_index("c")
    base = wid * BPW
    lanes = lax.broadcasted_iota(jnp.int32, (L,), 0)

    for c in range(BPW // 128):
        pltpu.sync_copy(ustage.at[pl.ds(base + c * 128, 128)], uv)
        pltpu.sync_copy(istage.at[pl.ds(base + c * 128, 128)], iv)
        for g in range(128 // L):
            rows = g * L + lanes
            acc = jnp.zeros((L,), jnp.float32)
            for d in range(D):
                dv = jnp.full((L,), d, jnp.int32)
                acc = acc + (plsc.load_gather(uv, [rows, dv])
                             * plsc.load_gather(iv, [rows, dv]))
            out_v[pl.ds(c * 128 + g * L, L)] = acc

    pltpu.sync_copy(out_v, out_hbm.at[pl.ds(base, BPW)])


def kernel(user_ids, item_ids, user_table, item_table):
    ustage, istage = _extract_kernel(
        user_ids.astype(jnp.int32), item_ids.astype(jnp.int32),
        user_table.T, item_table.T)
    return _dot_kernel(ustage, istage)


# speedup vs baseline: 23.1610x; 1.1848x over previous
import functools

import jax
import jax.numpy as jnp
from jax import lax
from jax.experimental import pallas as pl
from jax.experimental.pallas import tpu as pltpu
from jax.experimental.pallas import tpu_sc as plsc

BATCH = 16384
D = 64
VOCAB = 1000000

_info = plsc.get_sparse_core_info()
NC, NS, L = _info.num_cores, _info.num_subcores, _info.num_lanes
NW = NC * NS
BPW = BATCH // NW

NBLK = (VOCAB + 127) // 128
LASTK = NBLK - 1
NJ = 245
CAP = 16
GR = 4
LCAP = 768
NSL = BATCH // L
WDEPTH = 4
DUMPBASE = BATCH
STAGE_ROWS = BATCH + NW * L
SW = 128


def _filter_ids(allids_v, loc_ids, loc_pos, widv, lanes):
    def body(s, goffv):
        ids = allids_v[pl.ds(s * L, L)]
        mine = ((ids >> 7) & (NW - 1)) == widv
        csum = plsc.cumsum(mine.astype(jnp.int32))
        slot = goffv + csum - 1
        keep = mine & (slot >= 0) & (slot < LCAP)
        slot = jnp.where(keep, slot, 0)
        plsc.store_scatter(loc_ids, [slot], ids, mask=keep)
        plsc.store_scatter(loc_pos, [slot], s * L + lanes, mask=keep)
        return goffv + plsc.all_reduce_population_count(mine)
    return lax.fori_loop(0, NSL, body, jnp.zeros((L,), jnp.int32))


def _bin_ids(loc_ids, loc_pos, b_ids, b_pos, counts_v, goffv, lanes):
    def body(s2, carry):
        ids = loc_ids[pl.ds(s2 * L, L)]
        pos = loc_pos[pl.ds(s2 * L, L)]
        valid = (s2 * L + lanes) < goffv
        vi = valid.astype(jnp.int32)
        blkj = jnp.where(valid, ids >> 12, 0)
        rank = jnp.zeros((L,), jnp.int32)
        haslater = jnp.zeros((L,), jnp.int32)
        for s in range(1, L):
            idxb = jnp.maximum(lanes - s, 0)
            eqb = (jnp.take(blkj, idxb) == blkj) & (lanes >= s)
            rank = rank + (eqb & (jnp.take(vi, idxb) > 0)).astype(jnp.int32)
            idxf = jnp.minimum(lanes + s, L - 1)
            eqf = (jnp.take(blkj, idxf) == blkj) & (lanes + s <= L - 1)
            haslater = haslater + (eqf & (jnp.take(vi, idxf) > 0)).astype(jnp.int32)
        base = plsc.load_gather(counts_v, [blkj])
        slot = base + rank
        keep = valid & (slot < CAP)
        slot_s = jnp.where(keep, slot, 0)
        plsc.store_scatter(b_ids, [blkj, slot_s], ids, mask=keep)
        plsc.store_scatter(b_pos, [blkj, slot_s], pos, mask=keep)
        islast = valid & (haslater == 0)
        plsc.store_scatter(counts_v, [blkj], jnp.minimum(slot + 1, CAP),
                           mask=islast)
        return carry
    lax.fori_loop(0, LCAP // L, body, 0)


def _one_table_pass(ids_hbm, tab_hbm, stage_hbm, wid, widv, lanes,
                    allids_v, loc_ids, loc_pos, b_ids, b_pos, counts_v,
                    winbuf, rowtmp, posmat, counts_s, wsems, ssems):
    pltpu.sync_copy(ids_hbm, allids_v)
    for i in range(256 // L):
        counts_v[pl.ds(i * L, L)] = jnp.zeros((L,), jnp.int32)
    goffv = _filter_ids(allids_v, loc_ids, loc_pos, widv, lanes)
    _bin_ids(loc_ids, loc_pos, b_ids, b_pos, counts_v, goffv, lanes)

    def fire(jf, b):
        @pl.when(counts_s[jf] > 0)
        def _():
            kf = jnp.minimum(wid + 32 * jf, LASTK)
            koff = pl.multiple_of(kf * 128, 128)
            pltpu.async_copy(tab_hbm.at[:, pl.ds(koff, 128)],
                             winbuf.at[b], wsems[b])

    def drain_win(b):
        pltpu.make_async_copy(tab_hbm.at[:, pl.ds(0, 128)],
                              winbuf.at[b], wsems[b]).wait()

    def drain_scat(jd, b):
        @pl.when(counts_s[jd] > 0)
        def _():
            pltpu.make_async_copy(stage_hbm.at[pl.ds(0, CAP)],
                                  rowtmp.at[b], ssems[b]).wait()

    def extract(j, b, b2, cnt):
        cntv = jnp.full((L,), 0, jnp.int32) + cnt
        ids16 = b_ids[j]
        pos16 = b_pos[j]
        rm = ids16 & 127
        posmat[b2] = jnp.where(lanes < cntv, pos16,
                               DUMPBASE + widv * L + lanes)
        for j2 in range(CAP):

            @pl.when(cnt > j2)
            def _():
                rmb = jnp.take(rm, jnp.full((L,), j2, jnp.int32))
                for q in range(D // L):
                    vals = plsc.load_gather(winbuf.at[b], [q * L + lanes, rmb])
                    rowtmp[b2, j2, pl.ds(q * L, L)] = vals
        pltpu.async_copy(rowtmp.at[b2], stage_hbm.at[posmat.at[b2]],
                         ssems[b2])

    for i in range(256 // L):
        cv = counts_v[pl.ds(i * L, L)]
        for l in range(L):
            counts_s[i * L + l] = cv[l]

    for jp in range(WDEPTH):
        fire(jp, jp)

    def outer(j2, carry):
        for b in range(WDEPTH):
            j = WDEPTH * j2 + b
            b2 = b % 2

            @pl.when(j < NJ)
            def _():
                cnt = counts_s[j]

                @pl.when(cnt > 0)
                def _():
                    drain_win(b)

                @pl.when(j >= 2)
                def _():
                    drain_scat(j - 2, b2)

                @pl.when(cnt > 0)
                def _():
                    extract(j, b, b2, cnt)

                @pl.when(j + WDEPTH < NJ)
                def _():
                    fire(j + WDEPTH, b)
        return carry

    lax.fori_loop(0, (NJ + WDEPTH - 1) // WDEPTH, outer, 0)
    drain_scat(NJ - 2, (NJ - 2) % 2)
    drain_scat(NJ - 1, (NJ - 1) % 2)


@functools.partial(
    pl.kernel,
    out_type=(jax.ShapeDtypeStruct((STAGE_ROWS, SW), jnp.float32),
              jax.ShapeDtypeStruct((STAGE_ROWS, SW), jnp.float32)),
    mesh=plsc.VectorSubcoreMesh(core_axis_name="c", subcore_axis_name="s"),
    compiler_params=pltpu.CompilerParams(needs_layout_passes=False),
    scratch_types=[
        pltpu.VMEM((BATCH,), jnp.int32),
        pltpu.VMEM((LCAP,), jnp.int32),
        pltpu.VMEM((LCAP,), jnp.int32),
        pltpu.VMEM((NJ, CAP), jnp.int32),
        pltpu.VMEM((NJ, CAP), jnp.int32),
        pltpu.VMEM((256,), jnp.int32),
        pltpu.VMEM((WDEPTH, D, 128), jnp.float32),
        pltpu.VMEM((2, CAP, SW), jnp.float32),
        pltpu.VMEM((2, CAP), jnp.int32),
        pltpu.SMEM((256,), jnp.int32),
        pltpu.SemaphoreType.DMA,
        pltpu.SemaphoreType.DMA,
        pltpu.SemaphoreType.DMA,
        pltpu.SemaphoreType.DMA,
        pltpu.SemaphoreType.DMA,
        pltpu.SemaphoreType.DMA,
    ],
)
def _extract_kernel(uid_hbm, iid_hbm, ut_hbm, it_hbm, ustage, istage,
                    allids_v, loc_ids, loc_pos, b_ids, b_pos, counts_v,
                    winbuf, rowtmp, posmat, counts_s,
                    ws0, ws1, ws2, ws3, ss0, ss1):
    wid = lax.axis_index("s") * NC + lax.axis_index("c")
    widv = jnp.full((L,), 0, jnp.int32) + wid
    lanes = lax.broadcasted_iota(jnp.int32, (L,), 0)
    _one_table_pass(uid_hbm, ut_hbm, ustage, wid, widv, lanes,
                    allids_v, loc_ids, loc_pos, b_ids, b_pos, counts_v,
                    winbuf, rowtmp, posmat, counts_s,
                    (ws0, ws1, ws2, ws3), (ss0, ss1))
    _one_table_pass(iid_hbm, it_hbm, istage, wid, widv, lanes,
                    allids_v, loc_ids, loc_pos, b_ids, b_pos, counts_v,
                    winbuf, rowtmp, posmat, counts_s,
                    (ws0, ws1, ws2, ws3), (ss0, ss1))


@functools.partial(
    pl.kernel,
    out_type=jax.ShapeDtypeStruct((BATCH,), jnp.float32),
    mesh=plsc.VectorSubcoreMesh(core_axis_name="c", subcore_axis_name="s"),
    compiler_params=pltpu.CompilerParams(needs_layout_passes=False),
    scratch_types=[
        pltpu.VMEM((128, SW), jnp.float32),
        pltpu.VMEM((128, SW), jnp.float32),
        pltpu.VMEM((BPW,), jnp.float32),
    ],
)
def _dot_kernel(ustage, istage, out_hbm, uv, iv, out_v):
    wid = lax.axis_index("s") * NC + lax.axis_index("c")
    base = wid * BPW
    lanes = lax.broadcasted_iota(jnp.int32, (L,), 0)

    for c in range(BPW // 128):
        pltpu.sync_copy(ustage.at[pl.ds(base + c * 128, 128)], uv)
        pltpu.sync_copy(istage.at[pl.ds(base + c * 128, 128)], iv)
        for g in range(128 // L):
            rows = g * L + lanes
            acc = jnp.zeros((L,), jnp.float32)
            for d in range(D):
                dv = jnp.full((L,), d, jnp.int32)
                acc = acc + (plsc.load_gather(uv, [rows, dv])
                             * plsc.load_gather(iv, [rows, dv]))
            out_v[pl.ds(c * 128 + g * L, L)] = acc

    pltpu.sync_copy(out_v, out_hbm.at[pl.ds(base, BPW)])


def kernel(user_ids, item_ids, user_table, item_table):
    ustage, istage = _extract_kernel(
        user_ids.astype(jnp.int32), item_ids.astype(jnp.int32),
        user_table.T, item_table.T)
    return _dot_kernel(ustage, istage)
